# baseline (speedup 1.0000x reference)
import functools

import jax
import jax.numpy as jnp
from jax.experimental import pallas as pl
from jax.experimental.pallas import tpu as pltpu

ALPHA = 0.25
BN = 2048


def _matcher_body(p_ref, lp_ref, t_ref, lt_ref, amin_ref, cid_ref,
                  bv_ref, bi_ref, *, num_blocks):
    nb = pl.program_id(1)
    C = p_ref.shape[1]
    bn = p_ref.shape[2]
    M = t_ref.shape[1]

    @pl.when(nb == 0)
    def _init():
        bv_ref[...] = jnp.full((M, bn), jnp.inf, jnp.float32)
        bi_ref[...] = jnp.zeros((M, bn), jnp.int32)

    p = p_ref[0]
    neg_cost = (1.0 - ALPHA) * (p * p) * -jnp.log(1.0 - p + 1e-08)
    one_m_p = 1.0 - p
    pos_cost = ALPHA * (one_m_p * one_m_p) * -jnp.log(p + 1e-08)
    w = pos_cost - neg_cost

    t = (t_ref[0] == 1.0).astype(jnp.float32)
    cls_loss = jax.lax.dot_general(
        t, w, (((1,), (0,)), ((), ())),
        precision=jax.lax.Precision.HIGHEST)

    lp = lp_ref[0]
    lt = lt_ref[0]
    p_ymin, p_xmin = lp[0:1, :], lp[1:2, :]
    p_ymax, p_xmax = lp[2:3, :], lp[3:4, :]
    t_ymin, t_xmin = lt[:, 0:1], lt[:, 1:2]
    t_ymax, t_xmax = lt[:, 2:3], lt[:, 3:4]

    reg_loss = (((jnp.abs(t_ymin - p_ymin) + jnp.abs(t_xmin - p_xmin))
                 + jnp.abs(t_ymax - p_ymax)) + jnp.abs(t_xmax - p_xmax))

    pe_y = jnp.maximum(p_ymax - p_ymin, 0.0)
    pe_x = jnp.maximum(p_xmax - p_xmin, 0.0)
    te_y = jnp.maximum(t_ymax - t_ymin, 0.0)
    te_x = jnp.maximum(t_xmax - t_xmin, 0.0)
    b1_area = pe_y * pe_x
    b2_area = te_y * te_x
    i_ymin = jnp.maximum(p_ymin, t_ymin)
    i_xmin = jnp.maximum(p_xmin, t_xmin)
    i_ymax = jnp.minimum(p_ymax, t_ymax)
    i_xmax = jnp.minimum(p_xmax, t_xmax)
    d_y = i_ymax - i_ymin
    d_x = i_xmax - i_xmin
    inter = jnp.maximum(d_y, 0.0) * jnp.maximum(d_x, 0.0)
    union = b1_area + b2_area - inter
    iou = inter / jnp.where(union > 0.0, union, 1.0)
    enc = ((pe_y + te_y) - d_y) * ((pe_x + te_x) - d_x)
    giou_corr = (enc - union) / jnp.where(enc > 0.0, enc, 1.0)

    total = ((cls_loss + 2.5 * reg_loss) + (1.0 - iou)) + giou_corr

    bv = bv_ref[...]
    better = total < bv
    bv_ref[...] = jnp.minimum(total, bv)
    bi_ref[...] = jnp.where(better, nb, bi_ref[...])

    @pl.when(nb == num_blocks - 1)
    def _finish():
        bv = bv_ref[...]
        lane = jax.lax.broadcasted_iota(jnp.int32, (M, bn), 1)
        gidx = bi_ref[...] * bn + lane
        mv = jnp.min(bv, axis=1, keepdims=True)
        cand = jnp.where(bv == mv, gidx, jnp.int32(2 ** 30))
        amin_ref[0, :, :] = jnp.min(cand, axis=1, keepdims=True)
        tt = (t_ref[0] == 1.0).astype(jnp.float32)
        tmax = jnp.max(tt, axis=1, keepdims=True)
        ciota = jax.lax.broadcasted_iota(jnp.int32, (M, C), 1)
        cid = jnp.min(jnp.where(tt == tmax, ciota, jnp.int32(C)), axis=1, keepdims=True)
        cid_ref[0, :, :] = cid


def kernel(cls_pred, loc_pred, cls_true, loc_true, reg_mask):
    B, N, C = cls_pred.shape
    M = cls_true.shape[1]
    num_blocks = N // BN

    cls_pred_t = jnp.transpose(cls_pred, (0, 2, 1))
    loc_pred_t = jnp.transpose(loc_pred, (0, 2, 1))

    amin, cid = pl.pallas_call(
        functools.partial(_matcher_body, num_blocks=num_blocks),
        grid=(B, num_blocks),
        in_specs=[
            pl.BlockSpec((1, C, BN), lambda b, nb: (b, 0, nb)),
            pl.BlockSpec((1, 4, BN), lambda b, nb: (b, 0, nb)),
            pl.BlockSpec((1, M, C), lambda b, nb: (b, 0, 0)),
            pl.BlockSpec((1, M, 4), lambda b, nb: (b, 0, 0)),
        ],
        out_specs=[
            pl.BlockSpec((1, M, 1), lambda b, nb: (b, 0, 0)),
            pl.BlockSpec((1, M, 1), lambda b, nb: (b, 0, 0)),
        ],
        out_shape=[
            jax.ShapeDtypeStruct((B, M, 1), jnp.int32),
            jax.ShapeDtypeStruct((B, M, 1), jnp.int32),
        ],
        scratch_shapes=[
            pltpu.VMEM((M, BN), jnp.float32),
            pltpu.VMEM((M, BN), jnp.int32),
        ],
    )(cls_pred_t, loc_pred_t, cls_true, loc_true)

    batch = jnp.tile(jnp.arange(B, dtype=jnp.int32)[:, None], (1, M))
    return jnp.stack([batch, amin[:, :, 0], cid[:, :, 0]], axis=-1)


# speedup vs baseline: 1.1518x; 1.1518x over previous
import functools

import jax
import jax.numpy as jnp
from jax.experimental import pallas as pl
from jax.experimental.pallas import tpu as pltpu

ALPHA = 0.25
BN = 4096


def _matcher_body(p_ref, lp_ref, t_ref, lt_ref, amin_ref, cid_ref,
                  bv_ref, bi_ref, *, num_blocks):
    nb = pl.program_id(1)
    C = p_ref.shape[1]
    bn = p_ref.shape[2]
    M = t_ref.shape[1]

    @pl.when(nb == 0)
    def _init():
        bv_ref[...] = jnp.full((M, bn), jnp.inf, jnp.float32)
        bi_ref[...] = jnp.zeros((M, bn), jnp.int32)

    p = p_ref[0]
    neg_cost = (1.0 - ALPHA) * (p * p) * -jnp.log(1.0 - p + 1e-08)
    one_m_p = 1.0 - p
    pos_cost = ALPHA * (one_m_p * one_m_p) * -jnp.log(p + 1e-08)
    w = pos_cost - neg_cost

    t = (t_ref[0] == 1.0).astype(jnp.float32)
    cls_loss = jax.lax.dot_general(
        t, w, (((1,), (0,)), ((), ())),
        precision=jax.lax.Precision.HIGHEST)

    lp = lp_ref[0]
    lt = lt_ref[0]
    p_ymin, p_xmin = lp[0:1, :], lp[1:2, :]
    p_ymax, p_xmax = lp[2:3, :], lp[3:4, :]
    t_ymin, t_xmin = lt[:, 0:1], lt[:, 1:2]
    t_ymax, t_xmax = lt[:, 2:3], lt[:, 3:4]

    reg_loss = (((jnp.abs(t_ymin - p_ymin) + jnp.abs(t_xmin - p_xmin))
                 + jnp.abs(t_ymax - p_ymax)) + jnp.abs(t_xmax - p_xmax))

    pe_y = jnp.maximum(p_ymax - p_ymin, 0.0)
    pe_x = jnp.maximum(p_xmax - p_xmin, 0.0)
    te_y = jnp.maximum(t_ymax - t_ymin, 0.0)
    te_x = jnp.maximum(t_xmax - t_xmin, 0.0)
    b1_area = pe_y * pe_x
    b2_area = te_y * te_x
    i_ymin = jnp.maximum(p_ymin, t_ymin)
    i_xmin = jnp.maximum(p_xmin, t_xmin)
    i_ymax = jnp.minimum(p_ymax, t_ymax)
    i_xmax = jnp.minimum(p_xmax, t_xmax)
    d_y = i_ymax - i_ymin
    d_x = i_xmax - i_xmin
    inter = jnp.maximum(d_y, 0.0) * jnp.maximum(d_x, 0.0)
    union = b1_area + b2_area - inter
    iou = inter / jnp.where(union > 0.0, union, 1.0)
    enc = ((pe_y + te_y) - d_y) * ((pe_x + te_x) - d_x)
    giou_corr = (enc - union) / jnp.where(enc > 0.0, enc, 1.0)

    total = ((cls_loss + 2.5 * reg_loss) + (1.0 - iou)) + giou_corr

    bv = bv_ref[...]
    better = total < bv
    bv_ref[...] = jnp.minimum(total, bv)
    bi_ref[...] = jnp.where(better, nb, bi_ref[...])

    @pl.when(nb == num_blocks - 1)
    def _finish():
        bv = bv_ref[...]
        lane = jax.lax.broadcasted_iota(jnp.int32, (M, bn), 1)
        gidx = bi_ref[...] * bn + lane
        mv = jnp.min(bv, axis=1, keepdims=True)
        cand = jnp.where(bv == mv, gidx, jnp.int32(2 ** 30))
        amin_ref[0, :, :] = jnp.min(cand, axis=1, keepdims=True)
        tt = (t_ref[0] == 1.0).astype(jnp.float32)
        tmax = jnp.max(tt, axis=1, keepdims=True)
        ciota = jax.lax.broadcasted_iota(jnp.int32, (M, C), 1)
        cid = jnp.min(jnp.where(tt == tmax, ciota, jnp.int32(C)), axis=1, keepdims=True)
        cid_ref[0, :, :] = cid


def kernel(cls_pred, loc_pred, cls_true, loc_true, reg_mask):
    B, N, C = cls_pred.shape
    M = cls_true.shape[1]
    num_blocks = N // BN

    cls_pred_t = jnp.transpose(cls_pred, (0, 2, 1))
    loc_pred_t = jnp.transpose(loc_pred, (0, 2, 1))

    amin, cid = pl.pallas_call(
        functools.partial(_matcher_body, num_blocks=num_blocks),
        grid=(B, num_blocks),
        in_specs=[
            pl.BlockSpec((1, C, BN), lambda b, nb: (b, 0, nb)),
            pl.BlockSpec((1, 4, BN), lambda b, nb: (b, 0, nb)),
            pl.BlockSpec((1, M, C), lambda b, nb: (b, 0, 0)),
            pl.BlockSpec((1, M, 4), lambda b, nb: (b, 0, 0)),
        ],
        out_specs=[
            pl.BlockSpec((1, M, 1), lambda b, nb: (b, 0, 0)),
            pl.BlockSpec((1, M, 1), lambda b, nb: (b, 0, 0)),
        ],
        out_shape=[
            jax.ShapeDtypeStruct((B, M, 1), jnp.int32),
            jax.ShapeDtypeStruct((B, M, 1), jnp.int32),
        ],
        scratch_shapes=[
            pltpu.VMEM((M, BN), jnp.float32),
            pltpu.VMEM((M, BN), jnp.int32),
        ],
    )(cls_pred_t, loc_pred_t, cls_true, loc_true)

    batch = jnp.tile(jnp.arange(B, dtype=jnp.int32)[:, None], (1, M))
    return jnp.stack([batch, amin[:, :, 0], cid[:, :, 0]], axis=-1)
